# baseline (speedup 1.0000x reference)
import functools

import jax
import jax.numpy as jnp
from jax import lax
from jax.experimental import pallas as pl
from jax.experimental.pallas import tpu as pltpu
from jax.experimental.pallas import tpu_sc as plsc

N = 1_000_000
L = 16
NC = 2
NS = 16
NW = NC * NS
VECS = N // L
BASE_VECS = VECS // NW
REM = VECS - NW * BASE_VECS
U = 4
PAD_VECS = 1956
PIECES = (244, 244, 244, 244, 244, 244, 244, 248)
BUF = PAD_VECS * L

_mesh = plsc.VectorSubcoreMesh(core_axis_name="c", subcore_axis_name="s")


@functools.partial(
    pl.kernel,
    mesh=_mesh,
    out_type=jax.ShapeDtypeStruct((NW, 2 * L), jnp.float32),
    scratch_types=[
        pltpu.VMEM((BUF,), jnp.float32),
        pltpu.VMEM((2 * L,), jnp.float32),
    ] + [pltpu.SemaphoreType.DMA] * (len(PIECES) + 1),
)
def _margin_partials(score_hbm, out_hbm, buf, stage, *sems):
    c = lax.axis_index("c")
    s = lax.axis_index("s")
    wid = s * NC + c
    base_vec = wid * BASE_VECS + jnp.minimum(wid, REM)
    base = base_vec * L

    copies = []
    lo = 0
    for k, nvec in enumerate(PIECES):
        real = min(BASE_VECS, lo + nvec) - lo
        copies.append(
            pltpu.async_copy(
                score_hbm.at[pl.ds(base + lo * L, real * L)],
                buf.at[pl.ds(lo * L, real * L)],
                sems[k],
            )
        )
        lo += nvec

    extra_src = score_hbm.at[pl.ds(base + BASE_VECS * L, L)]
    extra_dst = buf.at[pl.ds(BASE_VECS * L, L)]

    @pl.when(wid < REM)
    def _():
        pltpu.async_copy(extra_src, extra_dst, sems[len(PIECES)])

    zf = jnp.zeros((L,), jnp.float32)
    for pad_vec in range(BASE_VECS + 1, PAD_VECS):
        buf[pl.ds(pad_vec * L, L)] = zf

    @pl.when(wid >= REM)
    def _():
        buf[pl.ds(BASE_VECS * L, L)] = zf

    def piece(lo, hi, carry):
        @plsc.parallel_loop(lo, hi, step=U, unroll=2, carry=carry)
        def body(i, accs):
            vss = list(accs[:U])
            vcs = list(accs[U:])
            for u in range(U):
                v = buf[pl.ds((i + u) * L, L)]
                vss[u] = vss[u] + jnp.minimum(v, 0.0)
                vcs[u] = vcs[u] + jnp.where(v < 0.0, 1.0, 0.0)
            return (*vss, *vcs)

        return body

    accs = (zf,) * (2 * U)
    lo = 0
    for k, nvec in enumerate(PIECES):
        copies[k].wait()
        if lo + nvec >= BASE_VECS:
            @pl.when(wid < REM)
            def _():
                pltpu.make_async_copy(extra_src, extra_dst, sems[len(PIECES)]).wait()

        accs = piece(lo, lo + nvec, accs)
        lo += nvec

    vs = (accs[0] + accs[1]) + (accs[2] + accs[3])
    vc = (accs[4] + accs[5]) + (accs[6] + accs[7])

    stage[pl.ds(0, L)] = vs
    stage[pl.ds(L, L)] = vc
    pltpu.sync_copy(stage, out_hbm.at[wid])


def kernel(preds, lables, score):
    out = _margin_partials(score)
    total = jnp.sum(out[:, :L])
    count = jnp.sum(out[:, L:])
    return total / count
